# Initial kernel scaffold; baseline (speedup 1.0000x reference)
import jax
import jax.numpy as jnp
from jax.experimental import pallas as pl


def kernel(conv1_w, conv1_b, conv2_w, conv2_b, conv3_w, conv3_b, fc1_w, fc1_b, fc2_w, fc2_b, x):
    raise NotImplementedError("write your pallas kernel here")


# speedup vs baseline: 16.8001x; 16.8001x over previous
import jax
import jax.numpy as jnp
from jax.experimental import pallas as pl
from jax.experimental.pallas import tpu as pltpu

_TB = 256


def _net_kernel(x_ref, w1_ref, w2_ref, w3_ref, wf1_ref, wf2_ref,
                b1_ref, b2_ref, b3_ref, bf1_ref, bf2_ref, out_ref,
                out1, p1, a2, out3):
    TB = out_ref.shape[-1]
    f32 = jnp.float32

    for g in range(6):
        rhs = x_ref[4 * g:4 * g + 7].reshape(224, TB)
        r = jnp.dot(w1_ref[...], rhs, preferred_element_type=f32)
        out1[4 * g:4 * g + 4] = r.reshape(4, 24, 8, TB)

    t = out1[...]
    t = jnp.max(t.reshape(12, 2, 24, 8, TB), axis=1)
    t = jnp.max(t.reshape(12, 12, 2, 8, TB), axis=2)
    t = jnp.maximum(t + b1_ref[...][None, None], 0.0)
    p1[...] = t.reshape(12, 96, TB).astype(jnp.bfloat16)

    for g in range(5):
        rhs = p1[2 * g:2 * g + 4].reshape(384, TB)
        r = jnp.dot(w2_ref[...], rhs, preferred_element_type=f32)
        r = jnp.maximum(r + b2_ref[...], 0.0)
        a2[2 * g:2 * g + 2] = r.reshape(2, 160, TB).astype(jnp.bfloat16)

    for h in range(8):
        rhs = a2[h:h + 3].reshape(480, TB)
        r = jnp.dot(w3_ref[...], rhs, preferred_element_type=f32)
        out3[h] = r.reshape(8, 16, TB)

    t3 = out3[...]
    t3 = jnp.max(t3.reshape(4, 2, 8, 16, TB), axis=1)
    t3 = jnp.max(t3.reshape(4, 4, 2, 16, TB), axis=2)
    t3 = jnp.maximum(t3 + b3_ref[...][None, None], 0.0)
    z = t3.reshape(256, TB)

    h1 = jnp.dot(wf1_ref[...], z, preferred_element_type=f32)
    h1 = jnp.maximum(h1 + bf1_ref[...], 0.0)
    logits = jnp.dot(wf2_ref[...], h1, preferred_element_type=f32)
    logits = logits + bf2_ref[...]
    m = jnp.max(logits, axis=0, keepdims=True)
    s = logits - m
    out_ref[...] = s - jnp.log(jnp.sum(jnp.exp(s), axis=0, keepdims=True))


def _banded_weights(conv1_w, conv2_w, conv3_w):
    f32 = jnp.float32
    A1 = jnp.stack([jnp.eye(4, 7, k=kh, dtype=f32) for kh in range(4)])
    B1 = jnp.stack([jnp.eye(24, 32, k=kw, dtype=f32) for kw in range(4)])
    W1 = jnp.einsum('ahH,bwW,oab->hwoHW', A1, B1, conv1_w[:, 0])
    W1 = W1.reshape(768, 224).astype(jnp.bfloat16)
    A2 = jnp.stack([jnp.eye(2, 4, k=kh, dtype=f32) for kh in range(3)])
    B2 = jnp.stack([jnp.eye(10, 12, k=kw, dtype=f32) for kw in range(3)])
    W2 = jnp.einsum('ahH,bwW,ocab->hwoHWc', A2, B2, conv2_w)
    W2 = W2.reshape(320, 384).astype(jnp.bfloat16)
    B3 = jnp.stack([jnp.eye(8, 10, k=kw, dtype=f32) for kw in range(3)])
    W3 = jnp.einsum('bwW,ocab->woaWc', B3, conv3_w)
    W3 = W3.reshape(128, 480).astype(jnp.bfloat16)
    return W1, W2, W3


def kernel(conv1_w, conv1_b, conv2_w, conv2_b, conv3_w, conv3_b,
           fc1_w, fc1_b, fc2_w, fc2_b, x):
    B = x.shape[0]
    n_tiles = max(1, (B + _TB - 1) // _TB)
    Bpad = n_tiles * _TB

    xt = jnp.transpose(x.reshape(B, 28, 28), (1, 2, 0)).astype(jnp.bfloat16)
    xt = jnp.pad(xt, ((0, 0), (0, 4), (0, Bpad - B)))

    W1, W2, W3 = _banded_weights(conv1_w, conv2_w, conv3_w)
    wf1 = fc1_w.reshape(64, 16, 4, 4).transpose(0, 2, 3, 1).reshape(64, 256)
    b1 = conv1_b.reshape(8, 1)
    b2 = jnp.broadcast_to(conv2_b[None, None, :], (2, 10, 16)).reshape(320, 1)
    b3 = conv3_b.reshape(16, 1)
    bf1 = fc1_b.reshape(64, 1)
    bf2 = fc2_b.reshape(10, 1)

    out = pl.pallas_call(
        _net_kernel,
        out_shape=jax.ShapeDtypeStruct((10, Bpad), jnp.float32),
        grid=(n_tiles,),
        in_specs=[
            pl.BlockSpec((28, 32, _TB), lambda b: (0, 0, b)),
            pl.BlockSpec((768, 224), lambda b: (0, 0)),
            pl.BlockSpec((320, 384), lambda b: (0, 0)),
            pl.BlockSpec((128, 480), lambda b: (0, 0)),
            pl.BlockSpec((64, 256), lambda b: (0, 0)),
            pl.BlockSpec((10, 64), lambda b: (0, 0)),
            pl.BlockSpec((8, 1), lambda b: (0, 0)),
            pl.BlockSpec((320, 1), lambda b: (0, 0)),
            pl.BlockSpec((16, 1), lambda b: (0, 0)),
            pl.BlockSpec((64, 1), lambda b: (0, 0)),
            pl.BlockSpec((10, 1), lambda b: (0, 0)),
        ],
        out_specs=pl.BlockSpec((10, _TB), lambda b: (0, b)),
        scratch_shapes=[
            pltpu.VMEM((24, 24, 8, _TB), jnp.float32),
            pltpu.VMEM((12, 96, _TB), jnp.bfloat16),
            pltpu.VMEM((10, 160, _TB), jnp.bfloat16),
            pltpu.VMEM((8, 8, 16, _TB), jnp.float32),
        ],
        compiler_params=pltpu.CompilerParams(
            dimension_semantics=("parallel",),
            vmem_limit_bytes=32 * 1024 * 1024,
        ),
    )(xt, W1, W2, W3, wf1, fc2_w, b1, b2, b3, bf1, bf2)

    return out[:, :B].T
